# Initial kernel scaffold; baseline (speedup 1.0000x reference)
import jax
import jax.numpy as jnp
from jax.experimental import pallas as pl


def kernel(x, edge_index, batch, W1_0, b1_0, g1_0, be1_0, W2_0, b2_0, go_0, bo_0, W1_1, b1_1, g1_1, be1_1, W2_1, b2_1, go_1, bo_1):
    raise NotImplementedError("write your pallas kernel here")


# speedup vs baseline: 4.4964x; 4.4964x over previous
import functools

import jax
import jax.numpy as jnp
from jax import lax
from jax.experimental import pallas as pl
from jax.experimental.pallas import tpu as pltpu
from jax.experimental.pallas import tpu_sc as plsc

_N = 10000
_D = 128
_E = 320000
_G = 64
_NC = 2
_NS = 16
_NW = _NC * _NS
_EPT = _E // _NW
_CH = 80
_EIT = _EPT // _CH
_RCH = 200
_NRC = _N // _RCH
_RIT = -(-_NRC // _NS)

_BR = 1000
_NB = _N // _BR
_EPS = 1e-5


def _sc_agg(z, src, dst):
    mesh = plsc.VectorSubcoreMesh(core_axis_name="c", subcore_axis_name="s",
                                  num_cores=_NC, num_subcores=_NS)

    @functools.partial(
        pl.kernel, mesh=mesh,
        out_type=jax.ShapeDtypeStruct((_NC * _N, _D), jnp.float32),
        scratch_types=[
            pltpu.VMEM((_CH,), jnp.int32),
            pltpu.VMEM((_CH,), jnp.int32),
            pltpu.VMEM((_CH, _D), jnp.float32),
            pltpu.VMEM((_RCH, _D), jnp.float32),
            pltpu.VMEM_SHARED((_N, _D), jnp.float32),
            pltpu.SemaphoreType.DMA,
        ],
    )
    def k(z_hbm, src_hbm, dst_hbm, out_hbm, src_v, dst_v, rows_v, buf_v,
          agg_sh, sem):
        c = lax.axis_index("c")
        s = lax.axis_index("s")

        def ibody(i, carry):
            idx = s + _NS * i

            @pl.when(idx < _NRC)
            def _():
                r0 = idx * _RCH
                pltpu.sync_copy(z_hbm.at[pl.ds(r0, _RCH)], buf_v)
                pltpu.sync_copy(buf_v, agg_sh.at[pl.ds(r0, _RCH)])
            return carry
        lax.fori_loop(0, _RIT, ibody, 0)
        plsc.subcore_barrier()

        wid = c * _NS + s

        def ebody(j, carry):
            off = wid * _EPT + j * _CH
            pltpu.sync_copy(src_hbm.at[pl.ds(off, _CH)], src_v)
            pltpu.sync_copy(dst_hbm.at[pl.ds(off, _CH)], dst_v)
            pltpu.async_copy(z_hbm.at[src_v], rows_v, sem).wait()
            pltpu.sync_copy(rows_v, agg_sh.at[dst_v], add=True)
            return carry
        lax.fori_loop(0, _EIT, ebody, 0)
        plsc.subcore_barrier()

        def obody(i, carry):
            idx = s + _NS * i

            @pl.when(idx < _NRC)
            def _():
                r0 = idx * _RCH
                pltpu.sync_copy(agg_sh.at[pl.ds(r0, _RCH)], buf_v)
                pltpu.sync_copy(buf_v, out_hbm.at[pl.ds(c * _N + r0, _RCH)])
            return carry
        lax.fori_loop(0, _RIT, obody, 0)

    return k(z, src, dst).reshape(_NC, _N, _D)


def _dot(a, b):
    return lax.dot_general(a, b, (((1,), (1,)), ((), ())),
                           preferred_element_type=jnp.float32)


def _mlp1_body(z_ref, p_ref, w_ref, b_ref, h_ref, st_ref):
    i = pl.program_id(0)
    a = p_ref[0] + p_ref[1] - z_ref[...]
    h = _dot(a, w_ref[...]) + b_ref[...]
    h_ref[...] = h

    @pl.when(i == 0)
    def _():
        st_ref[...] = jnp.zeros_like(st_ref)
    st_ref[...] += jnp.concatenate(
        [jnp.sum(h, axis=0, keepdims=True),
         jnp.sum(h * h, axis=0, keepdims=True)], axis=0)


def _mlp1(z, p, w1, b1):
    return pl.pallas_call(
        _mlp1_body,
        grid=(_NB,),
        in_specs=[
            pl.BlockSpec((_BR, _D), lambda i: (i, 0)),
            pl.BlockSpec((_NC, _BR, _D), lambda i: (0, i, 0)),
            pl.BlockSpec((_D, _D), lambda i: (0, 0)),
            pl.BlockSpec((1, _D), lambda i: (0, 0)),
        ],
        out_specs=[
            pl.BlockSpec((_BR, _D), lambda i: (i, 0)),
            pl.BlockSpec((2, _D), lambda i: (0, 0)),
        ],
        out_shape=[
            jax.ShapeDtypeStruct((_N, _D), jnp.float32),
            jax.ShapeDtypeStruct((2, _D), jnp.float32),
        ],
    )(z, p, w1, b1)


def _bn_coeffs(st_ref, g_ref, be_ref):
    mu = st_ref[0:1, :] * (1.0 / _N)
    var = st_ref[1:2, :] * (1.0 / _N) - mu * mu
    inv = lax.rsqrt(var + _EPS)
    scale = g_ref[...] * inv
    shift = be_ref[...] - mu * scale
    return scale, shift


def _mlp2_body(h_ref, st_ref, g_ref, be_ref, w_ref, b_ref, t_ref, st2_ref):
    i = pl.program_id(0)
    scale, shift = _bn_coeffs(st_ref, g_ref, be_ref)
    hn = jnp.maximum(h_ref[...] * scale + shift, 0.0)
    t = jnp.maximum(_dot(hn, w_ref[...]) + b_ref[...], 0.0)
    t_ref[...] = t

    @pl.when(i == 0)
    def _():
        st2_ref[...] = jnp.zeros_like(st2_ref)
    st2_ref[...] += jnp.concatenate(
        [jnp.sum(t, axis=0, keepdims=True),
         jnp.sum(t * t, axis=0, keepdims=True)], axis=0)


def _mlp2(h, st, g1, be1, w2, b2):
    return pl.pallas_call(
        _mlp2_body,
        grid=(_NB,),
        in_specs=[
            pl.BlockSpec((_BR, _D), lambda i: (i, 0)),
            pl.BlockSpec((2, _D), lambda i: (0, 0)),
            pl.BlockSpec((1, _D), lambda i: (0, 0)),
            pl.BlockSpec((1, _D), lambda i: (0, 0)),
            pl.BlockSpec((_D, _D), lambda i: (0, 0)),
            pl.BlockSpec((1, _D), lambda i: (0, 0)),
        ],
        out_specs=[
            pl.BlockSpec((_BR, _D), lambda i: (i, 0)),
            pl.BlockSpec((2, _D), lambda i: (0, 0)),
        ],
        out_shape=[
            jax.ShapeDtypeStruct((_N, _D), jnp.float32),
            jax.ShapeDtypeStruct((2, _D), jnp.float32),
        ],
    )(h, st, g1, be1, w2, b2)


def _bnpool_body(t_ref, st_ref, g_ref, bo_ref, batch_ref, z_ref, gp_ref):
    i = pl.program_id(0)
    scale, shift = _bn_coeffs(st_ref, g_ref, bo_ref)
    z = t_ref[...] * scale + shift
    z_ref[...] = z

    b = batch_ref[0]
    onehot = (lax.broadcasted_iota(jnp.int32, (_G, _BR), 0) == b
              ).astype(jnp.float32)
    gp = lax.dot_general(onehot, z, (((1,), (0,)), ((), ())),
                         preferred_element_type=jnp.float32)

    @pl.when(i == 0)
    def _():
        gp_ref[...] = jnp.zeros_like(gp_ref)
    gp_ref[...] += gp


def _bnpool(t, st, go, bo, batch3):
    return pl.pallas_call(
        _bnpool_body,
        grid=(_NB,),
        in_specs=[
            pl.BlockSpec((_BR, _D), lambda i: (i, 0)),
            pl.BlockSpec((2, _D), lambda i: (0, 0)),
            pl.BlockSpec((1, _D), lambda i: (0, 0)),
            pl.BlockSpec((1, _D), lambda i: (0, 0)),
            pl.BlockSpec((1, 1, _BR), lambda i: (i, 0, 0)),
        ],
        out_specs=[
            pl.BlockSpec((_BR, _D), lambda i: (i, 0)),
            pl.BlockSpec((_G, _D), lambda i: (0, 0)),
        ],
        out_shape=[
            jax.ShapeDtypeStruct((_N, _D), jnp.float32),
            jax.ShapeDtypeStruct((_G, _D), jnp.float32),
        ],
    )(t, st, go, bo, batch3)


def kernel(x, edge_index, batch,
           W1_0, b1_0, g1_0, be1_0, W2_0, b2_0, go_0, bo_0,
           W1_1, b1_1, g1_1, be1_1, W2_1, b2_1, go_1, bo_1):
    src = edge_index[0]
    dst = edge_index[1]
    batch3 = batch.reshape(_NB, 1, _BR)
    params = [(W1_0, b1_0, g1_0, be1_0, W2_0, b2_0, go_0, bo_0),
              (W1_1, b1_1, g1_1, be1_1, W2_1, b2_1, go_1, bo_1)]

    z = x
    zs, gs = [], []
    for (W1, b1, g1, be1, W2, b2, go, bo) in params:
        p = _sc_agg(z, src, dst)
        h, st1 = _mlp1(z, p, W1, b1.reshape(1, _D))
        t, st2 = _mlp2(h, st1, g1.reshape(1, _D), be1.reshape(1, _D),
                       W2, b2.reshape(1, _D))
        z, g = _bnpool(t, st2, go.reshape(1, _D), bo.reshape(1, _D), batch3)
        zs.append(z)
        gs.append(g)

    return jnp.concatenate(zs, axis=1), jnp.concatenate(gs, axis=1)


# speedup vs baseline: 4.7486x; 1.0561x over previous
import functools

import jax
import jax.numpy as jnp
from jax import lax
from jax.experimental import pallas as pl
from jax.experimental.pallas import tpu as pltpu
from jax.experimental.pallas import tpu_sc as plsc

_N = 10000
_D = 128
_E = 320000
_G = 64
_NC = 2
_NS = 16
_NW = _NC * _NS
_DH = _D // _NC
_EPT = _E // _NS
_CH = 128
_EIT = 160
_EPP = _EIT * _CH
_NTR = 16
_RCH = 200
_NRC = _N // _RCH
_RIT = -(-_NRC // _NS)

_BR = 1000
_NB = _N // _BR
_EPS = 1e-5


def _sc_agg(zt, src, dst):
    mesh = plsc.VectorSubcoreMesh(core_axis_name="c", subcore_axis_name="s",
                                  num_cores=_NC, num_subcores=_NS)

    @functools.partial(
        pl.kernel, mesh=mesh,
        compiler_params=pltpu.CompilerParams(use_tc_tiling_on_sc=False),
        out_type=jax.ShapeDtypeStruct((_NC * _N, _DH), jnp.float32),
        scratch_types=[
            pltpu.VMEM((_EPP,), jnp.int32),
            pltpu.VMEM((_EIT, _CH), jnp.int32),
            pltpu.VMEM((_CH, _DH), jnp.float32),
            pltpu.VMEM((_CH, _DH), jnp.float32),
            pltpu.VMEM((_RCH, _DH), jnp.float32),
            pltpu.VMEM_SHARED((_N + _NTR, _DH), jnp.float32),
            pltpu.SemaphoreType.DMA,
            pltpu.SemaphoreType.DMA,
        ],
    )
    def k(zt_hbm, src_hbm, dst_hbm, out_hbm, src_v, dst_v, rows_a, rows_b,
          buf_v, agg_sh, sem_a, sem_b):
        c = lax.axis_index("c")
        s = lax.axis_index("s")
        zc = zt_hbm.at[c]

        pltpu.sync_copy(src_hbm.at[pl.ds(s * _EPP, _EPP)], src_v)
        pltpu.sync_copy(dst_hbm.at[pl.ds(s * _EIT, _EIT)], dst_v)

        def ibody(i, carry):
            idx = s + _NS * i

            @pl.when(idx < _NRC)
            def _():
                r0 = idx * _RCH
                pltpu.sync_copy(zc.at[pl.ds(r0, _RCH)], buf_v)
                pltpu.sync_copy(buf_v, agg_sh.at[pl.ds(r0, _RCH)])
            return carry
        lax.fori_loop(0, _RIT, ibody, 0)
        plsc.subcore_barrier()

        def issue(j, buf, sem):
            pltpu.async_copy(zc.at[src_v.at[pl.ds(j * _CH, _CH)]], buf, sem)

        def drain(buf, sem):
            pltpu.make_async_copy(zc.at[src_v.at[pl.ds(0, _CH)]], buf,
                                  sem).wait()

        def scatter(j, buf):
            pltpu.sync_copy(buf, agg_sh.at[dst_v.at[j]], add=True)

        issue(0, rows_a, sem_a)

        def ebody(j, carry):
            @pl.when(j % 2 == 0)
            def _():
                issue(j + 1, rows_b, sem_b)
                drain(rows_a, sem_a)
                scatter(j, rows_a)

            @pl.when(j % 2 == 1)
            def _():
                issue(j + 1, rows_a, sem_a)
                drain(rows_b, sem_b)
                scatter(j, rows_b)
            return carry
        lax.fori_loop(0, _EIT - 1, ebody, 0)
        drain(rows_b, sem_b)
        scatter(_EIT - 1, rows_b)
        plsc.subcore_barrier()

        def obody(i, carry):
            idx = s + _NS * i

            @pl.when(idx < _NRC)
            def _():
                r0 = idx * _RCH
                pltpu.sync_copy(agg_sh.at[pl.ds(r0, _RCH)], buf_v)
                pltpu.sync_copy(buf_v, out_hbm.at[pl.ds(c * _N + r0, _RCH)])
            return carry
        lax.fori_loop(0, _RIT, obody, 0)

    return k(zt, src, dst).reshape(_NC, _N, _DH)


def _dot(a, b):
    return lax.dot_general(a, b, (((1,), (1,)), ((), ())),
                           preferred_element_type=jnp.float32)


def _mlp1_body(p_ref, w_ref, b_ref, h_ref, st_ref):
    i = pl.program_id(0)
    a = jnp.concatenate([p_ref[0], p_ref[1]], axis=1)
    h = _dot(a, w_ref[...]) + b_ref[...]
    h_ref[...] = h

    @pl.when(i == 0)
    def _():
        st_ref[...] = jnp.zeros_like(st_ref)
    st_ref[...] += jnp.concatenate(
        [jnp.sum(h, axis=0, keepdims=True),
         jnp.sum(h * h, axis=0, keepdims=True)], axis=0)


def _mlp1(p, w1, b1):
    return pl.pallas_call(
        _mlp1_body,
        grid=(_NB,),
        in_specs=[
            pl.BlockSpec((_NC, _BR, _DH), lambda i: (0, i, 0)),
            pl.BlockSpec((_D, _D), lambda i: (0, 0)),
            pl.BlockSpec((1, _D), lambda i: (0, 0)),
        ],
        out_specs=[
            pl.BlockSpec((_BR, _D), lambda i: (i, 0)),
            pl.BlockSpec((2, _D), lambda i: (0, 0)),
        ],
        out_shape=[
            jax.ShapeDtypeStruct((_N, _D), jnp.float32),
            jax.ShapeDtypeStruct((2, _D), jnp.float32),
        ],
    )(p, w1, b1)


def _bn_coeffs(st_ref, g_ref, be_ref):
    mu = st_ref[0:1, :] * (1.0 / _N)
    var = st_ref[1:2, :] * (1.0 / _N) - mu * mu
    inv = lax.rsqrt(var + _EPS)
    scale = g_ref[...] * inv
    shift = be_ref[...] - mu * scale
    return scale, shift


def _mlp2_body(h_ref, st_ref, g_ref, be_ref, w_ref, b_ref, t_ref, st2_ref):
    i = pl.program_id(0)
    scale, shift = _bn_coeffs(st_ref, g_ref, be_ref)
    hn = jnp.maximum(h_ref[...] * scale + shift, 0.0)
    t = jnp.maximum(_dot(hn, w_ref[...]) + b_ref[...], 0.0)
    t_ref[...] = t

    @pl.when(i == 0)
    def _():
        st2_ref[...] = jnp.zeros_like(st2_ref)
    st2_ref[...] += jnp.concatenate(
        [jnp.sum(t, axis=0, keepdims=True),
         jnp.sum(t * t, axis=0, keepdims=True)], axis=0)


def _mlp2(h, st, g1, be1, w2, b2):
    return pl.pallas_call(
        _mlp2_body,
        grid=(_NB,),
        in_specs=[
            pl.BlockSpec((_BR, _D), lambda i: (i, 0)),
            pl.BlockSpec((2, _D), lambda i: (0, 0)),
            pl.BlockSpec((1, _D), lambda i: (0, 0)),
            pl.BlockSpec((1, _D), lambda i: (0, 0)),
            pl.BlockSpec((_D, _D), lambda i: (0, 0)),
            pl.BlockSpec((1, _D), lambda i: (0, 0)),
        ],
        out_specs=[
            pl.BlockSpec((_BR, _D), lambda i: (i, 0)),
            pl.BlockSpec((2, _D), lambda i: (0, 0)),
        ],
        out_shape=[
            jax.ShapeDtypeStruct((_N, _D), jnp.float32),
            jax.ShapeDtypeStruct((2, _D), jnp.float32),
        ],
    )(h, st, g1, be1, w2, b2)


def _bnpool_body(t_ref, st_ref, g_ref, bo_ref, batch_ref, z_ref, gp_ref):
    i = pl.program_id(0)
    scale, shift = _bn_coeffs(st_ref, g_ref, bo_ref)
    z = t_ref[...] * scale + shift
    z_ref[...] = z

    b = batch_ref[0]
    onehot = (lax.broadcasted_iota(jnp.int32, (_G, _BR), 0) == b
              ).astype(jnp.float32)
    gp = lax.dot_general(onehot, z, (((1,), (0,)), ((), ())),
                         preferred_element_type=jnp.float32)

    @pl.when(i == 0)
    def _():
        gp_ref[...] = jnp.zeros_like(gp_ref)
    gp_ref[...] += gp


def _bnpool(t, st, go, bo, batch3):
    return pl.pallas_call(
        _bnpool_body,
        grid=(_NB,),
        in_specs=[
            pl.BlockSpec((_BR, _D), lambda i: (i, 0)),
            pl.BlockSpec((2, _D), lambda i: (0, 0)),
            pl.BlockSpec((1, _D), lambda i: (0, 0)),
            pl.BlockSpec((1, _D), lambda i: (0, 0)),
            pl.BlockSpec((1, 1, _BR), lambda i: (i, 0, 0)),
        ],
        out_specs=[
            pl.BlockSpec((_BR, _D), lambda i: (i, 0)),
            pl.BlockSpec((_G, _D), lambda i: (0, 0)),
        ],
        out_shape=[
            jax.ShapeDtypeStruct((_N, _D), jnp.float32),
            jax.ShapeDtypeStruct((_G, _D), jnp.float32),
        ],
    )(t, st, go, bo, batch3)


def kernel(x, edge_index, batch,
           W1_0, b1_0, g1_0, be1_0, W2_0, b2_0, go_0, bo_0,
           W1_1, b1_1, g1_1, be1_1, W2_1, b2_1, go_1, bo_1):
    pad = _EPP - _EPT
    src = jnp.pad(edge_index[0].reshape(_NS, _EPT),
                  ((0, 0), (0, pad))).reshape(_NS * _EPP)
    dst = jnp.pad(edge_index[1].reshape(_NS, _EPT), ((0, 0), (0, pad)),
                  constant_values=_N).reshape(_NS * _EIT, _CH)
    batch3 = batch.reshape(_NB, 1, _BR)
    params = [(W1_0, b1_0, g1_0, be1_0, W2_0, b2_0, go_0, bo_0),
              (W1_1, b1_1, g1_1, be1_1, W2_1, b2_1, go_1, bo_1)]

    z = x
    zs, gs = [], []
    for (W1, b1, g1, be1, W2, b2, go, bo) in params:
        zt = jnp.moveaxis(z.reshape(_N, _NC, _DH), 1, 0)
        p = _sc_agg(zt, src, dst)
        h, st1 = _mlp1(p, W1, b1.reshape(1, _D))
        t, st2 = _mlp2(h, st1, g1.reshape(1, _D), be1.reshape(1, _D),
                       W2, b2.reshape(1, _D))
        z, g = _bnpool(t, st2, go.reshape(1, _D), bo.reshape(1, _D), batch3)
        zs.append(z)
        gs.append(g)

    return jnp.concatenate(zs, axis=1), jnp.concatenate(gs, axis=1)


# speedup vs baseline: 5.0776x; 1.0693x over previous
import functools

import jax
import jax.numpy as jnp
from jax import lax
from jax.experimental import pallas as pl
from jax.experimental.pallas import tpu as pltpu
from jax.experimental.pallas import tpu_sc as plsc

_N = 10000
_D = 128
_E = 320000
_G = 64
_NC = 2
_NS = 16
_NW = _NC * _NS
_DH = _D // _NC
_EPT = _E // _NS
_CH = 128
_EIT = 160
_EPP = _EIT * _CH
_NBF = 4
_NGR = _EIT // _NBF
_NTR = 16
_RCH = 200
_NRC = _N // _RCH
_RIT = -(-_NRC // _NS)

_BR = 1000
_NB = _N // _BR
_EPS = 1e-5


def _sc_agg(zt, src, dst):
    mesh = plsc.VectorSubcoreMesh(core_axis_name="c", subcore_axis_name="s",
                                  num_cores=_NC, num_subcores=_NS)

    @functools.partial(
        pl.kernel, mesh=mesh,
        compiler_params=pltpu.CompilerParams(use_tc_tiling_on_sc=False),
        out_type=jax.ShapeDtypeStruct((_NC * _N, _DH), jnp.float32),
        scratch_types=[
            pltpu.VMEM((_EPP,), jnp.int32),
            pltpu.VMEM((_EIT, _CH), jnp.int32),
            pltpu.VMEM((_NBF, _CH, _DH), jnp.float32),
            pltpu.VMEM((_RCH, _DH), jnp.float32),
            pltpu.VMEM_SHARED((_N + _NTR, _DH), jnp.float32),
            pltpu.SemaphoreType.DMA((_NBF,)),
            pltpu.SemaphoreType.DMA((_NBF,)),
        ],
    )
    def k(zt_hbm, src_hbm, dst_hbm, out_hbm, src_v, dst_v, rows_v,
          buf_v, agg_sh, gsem, ssem):
        c = lax.axis_index("c")
        s = lax.axis_index("s")
        zc = zt_hbm.at[c]

        pltpu.sync_copy(src_hbm.at[pl.ds(s * _EPP, _EPP)], src_v)
        pltpu.sync_copy(dst_hbm.at[pl.ds(s * _EIT, _EIT)], dst_v)

        def ibody(i, carry):
            idx = s + _NS * i

            @pl.when(idx < _NRC)
            def _():
                r0 = idx * _RCH
                pltpu.sync_copy(zc.at[pl.ds(r0, _RCH)], buf_v)
                pltpu.sync_copy(buf_v, agg_sh.at[pl.ds(r0, _RCH)])
            return carry
        lax.fori_loop(0, _RIT, ibody, 0)
        plsc.subcore_barrier()

        def issue_gather(j, b):
            pltpu.async_copy(zc.at[src_v.at[pl.ds(j * _CH, _CH)]],
                             rows_v.at[b], gsem.at[b])

        def wait_gather(b):
            pltpu.make_async_copy(zc.at[src_v.at[pl.ds(0, _CH)]],
                                  rows_v.at[b], gsem.at[b]).wait()

        def issue_scatter(j, b):
            pltpu.async_copy(rows_v.at[b], agg_sh.at[dst_v.at[j]],
                             ssem.at[b], add=True)

        def wait_scatter(b):
            pltpu.make_async_copy(rows_v.at[b], agg_sh.at[dst_v.at[0]],
                                  ssem.at[b]).wait()

        for b in range(_NBF):
            issue_gather(b, b)
        wait_gather(0)
        issue_scatter(0, 0)
        for b in range(1, _NBF):
            wait_gather(b)
            issue_scatter(b, b)
            wait_scatter(b - 1)
            issue_gather(b - 1 + _NBF, b - 1)

        def ebody(g, carry):
            for b in range(_NBF):
                jj = g * _NBF + b
                wait_gather(b)
                issue_scatter(jj, b)
                bp = (b - 1) % _NBF
                wait_scatter(bp)
                issue_gather(jj + _NBF - 1, bp)
            return carry
        lax.fori_loop(1, _NGR - 1, ebody, 0)

        j0 = (_NGR - 1) * _NBF
        wait_gather(0)
        issue_scatter(j0, 0)
        wait_scatter(_NBF - 1)
        issue_gather(j0 + _NBF - 1, _NBF - 1)
        for b in range(1, _NBF):
            wait_gather(b)
            issue_scatter(j0 + b, b)
        for b in range(_NBF):
            wait_scatter(b)
        plsc.subcore_barrier()

        def obody(i, carry):
            idx = s + _NS * i

            @pl.when(idx < _NRC)
            def _():
                r0 = idx * _RCH
                pltpu.sync_copy(agg_sh.at[pl.ds(r0, _RCH)], buf_v)
                pltpu.sync_copy(buf_v, out_hbm.at[pl.ds(c * _N + r0, _RCH)])
            return carry
        lax.fori_loop(0, _RIT, obody, 0)

    return k(zt, src, dst).reshape(_NC, _N, _DH)


def _dot(a, b):
    return lax.dot_general(a, b, (((1,), (1,)), ((), ())),
                           preferred_element_type=jnp.float32)


def _mlp1_body(p_ref, w_ref, b_ref, h_ref, st_ref):
    i = pl.program_id(0)
    a = jnp.concatenate([p_ref[0], p_ref[1]], axis=1)
    h = _dot(a, w_ref[...]) + b_ref[...]
    h_ref[...] = h

    @pl.when(i == 0)
    def _():
        st_ref[...] = jnp.zeros_like(st_ref)
    st_ref[...] += jnp.concatenate(
        [jnp.sum(h, axis=0, keepdims=True),
         jnp.sum(h * h, axis=0, keepdims=True)], axis=0)


def _mlp1(p, w1, b1):
    return pl.pallas_call(
        _mlp1_body,
        grid=(_NB,),
        in_specs=[
            pl.BlockSpec((_NC, _BR, _DH), lambda i: (0, i, 0)),
            pl.BlockSpec((_D, _D), lambda i: (0, 0)),
            pl.BlockSpec((1, _D), lambda i: (0, 0)),
        ],
        out_specs=[
            pl.BlockSpec((_BR, _D), lambda i: (i, 0)),
            pl.BlockSpec((2, _D), lambda i: (0, 0)),
        ],
        out_shape=[
            jax.ShapeDtypeStruct((_N, _D), jnp.float32),
            jax.ShapeDtypeStruct((2, _D), jnp.float32),
        ],
    )(p, w1, b1)


def _bn_coeffs(st_ref, g_ref, be_ref):
    mu = st_ref[0:1, :] * (1.0 / _N)
    var = st_ref[1:2, :] * (1.0 / _N) - mu * mu
    inv = lax.rsqrt(var + _EPS)
    scale = g_ref[...] * inv
    shift = be_ref[...] - mu * scale
    return scale, shift


def _mlp2_body(h_ref, st_ref, g_ref, be_ref, w_ref, b_ref, t_ref, st2_ref):
    i = pl.program_id(0)
    scale, shift = _bn_coeffs(st_ref, g_ref, be_ref)
    hn = jnp.maximum(h_ref[...] * scale + shift, 0.0)
    t = jnp.maximum(_dot(hn, w_ref[...]) + b_ref[...], 0.0)
    t_ref[...] = t

    @pl.when(i == 0)
    def _():
        st2_ref[...] = jnp.zeros_like(st2_ref)
    st2_ref[...] += jnp.concatenate(
        [jnp.sum(t, axis=0, keepdims=True),
         jnp.sum(t * t, axis=0, keepdims=True)], axis=0)


def _mlp2(h, st, g1, be1, w2, b2):
    return pl.pallas_call(
        _mlp2_body,
        grid=(_NB,),
        in_specs=[
            pl.BlockSpec((_BR, _D), lambda i: (i, 0)),
            pl.BlockSpec((2, _D), lambda i: (0, 0)),
            pl.BlockSpec((1, _D), lambda i: (0, 0)),
            pl.BlockSpec((1, _D), lambda i: (0, 0)),
            pl.BlockSpec((_D, _D), lambda i: (0, 0)),
            pl.BlockSpec((1, _D), lambda i: (0, 0)),
        ],
        out_specs=[
            pl.BlockSpec((_BR, _D), lambda i: (i, 0)),
            pl.BlockSpec((2, _D), lambda i: (0, 0)),
        ],
        out_shape=[
            jax.ShapeDtypeStruct((_N, _D), jnp.float32),
            jax.ShapeDtypeStruct((2, _D), jnp.float32),
        ],
    )(h, st, g1, be1, w2, b2)


def _bnpool_body(t_ref, st_ref, g_ref, bo_ref, batch_ref, z_ref, gp_ref):
    i = pl.program_id(0)
    scale, shift = _bn_coeffs(st_ref, g_ref, bo_ref)
    z = t_ref[...] * scale + shift
    z_ref[...] = z

    b = batch_ref[0]
    onehot = (lax.broadcasted_iota(jnp.int32, (_G, _BR), 0) == b
              ).astype(jnp.float32)
    gp = lax.dot_general(onehot, z, (((1,), (0,)), ((), ())),
                         preferred_element_type=jnp.float32)

    @pl.when(i == 0)
    def _():
        gp_ref[...] = jnp.zeros_like(gp_ref)
    gp_ref[...] += gp


def _bnpool(t, st, go, bo, batch3):
    return pl.pallas_call(
        _bnpool_body,
        grid=(_NB,),
        in_specs=[
            pl.BlockSpec((_BR, _D), lambda i: (i, 0)),
            pl.BlockSpec((2, _D), lambda i: (0, 0)),
            pl.BlockSpec((1, _D), lambda i: (0, 0)),
            pl.BlockSpec((1, _D), lambda i: (0, 0)),
            pl.BlockSpec((1, 1, _BR), lambda i: (i, 0, 0)),
        ],
        out_specs=[
            pl.BlockSpec((_BR, _D), lambda i: (i, 0)),
            pl.BlockSpec((_G, _D), lambda i: (0, 0)),
        ],
        out_shape=[
            jax.ShapeDtypeStruct((_N, _D), jnp.float32),
            jax.ShapeDtypeStruct((_G, _D), jnp.float32),
        ],
    )(t, st, go, bo, batch3)


def kernel(x, edge_index, batch,
           W1_0, b1_0, g1_0, be1_0, W2_0, b2_0, go_0, bo_0,
           W1_1, b1_1, g1_1, be1_1, W2_1, b2_1, go_1, bo_1):
    pad = _EPP - _EPT
    src = jnp.pad(edge_index[0].reshape(_NS, _EPT),
                  ((0, 0), (0, pad))).reshape(_NS * _EPP)
    dst = jnp.pad(edge_index[1].reshape(_NS, _EPT), ((0, 0), (0, pad)),
                  constant_values=_N).reshape(_NS * _EIT, _CH)
    batch3 = batch.reshape(_NB, 1, _BR)
    params = [(W1_0, b1_0, g1_0, be1_0, W2_0, b2_0, go_0, bo_0),
              (W1_1, b1_1, g1_1, be1_1, W2_1, b2_1, go_1, bo_1)]

    z = x
    zs, gs = [], []
    for (W1, b1, g1, be1, W2, b2, go, bo) in params:
        zt = jnp.moveaxis(z.reshape(_N, _NC, _DH), 1, 0)
        p = _sc_agg(zt, src, dst)
        h, st1 = _mlp1(p, W1, b1.reshape(1, _D))
        t, st2 = _mlp2(h, st1, g1.reshape(1, _D), be1.reshape(1, _D),
                       W2, b2.reshape(1, _D))
        z, g = _bnpool(t, st2, go.reshape(1, _D), bo.reshape(1, _D), batch3)
        zs.append(z)
        gs.append(g)

    return jnp.concatenate(zs, axis=1), jnp.concatenate(gs, axis=1)


# speedup vs baseline: 5.2211x; 1.0283x over previous
import functools

import jax
import jax.numpy as jnp
from jax import lax
from jax.experimental import pallas as pl
from jax.experimental.pallas import tpu as pltpu
from jax.experimental.pallas import tpu_sc as plsc

_N = 10000
_D = 128
_E = 320000
_G = 64
_NC = 2
_NS = 16
_NW = _NC * _NS
_DH = _D // _NC
_EPT = _E // _NS
_CH = 128
_EIT = 160
_EPP = _EIT * _CH
_NBF = 8
_NGR = _EIT // _NBF
_NPAR = 3
_NTR = 16
_RCH = 200
_NRC = _N // _RCH
_RIT = -(-_NRC // _NS)

_BR = 1000
_NB = _N // _BR
_EPS = 1e-5


def _sc_agg(zt, src, dst):
    mesh = plsc.VectorSubcoreMesh(core_axis_name="c", subcore_axis_name="s",
                                  num_cores=_NC, num_subcores=_NS)

    @functools.partial(
        pl.kernel, mesh=mesh,
        compiler_params=pltpu.CompilerParams(use_tc_tiling_on_sc=False),
        out_type=jax.ShapeDtypeStruct((_NC * _N, _DH), jnp.float32),
        scratch_types=[
            pltpu.VMEM((_NPAR, _NBF * _CH), jnp.int32),
            pltpu.VMEM((_NPAR, _NBF, _CH), jnp.int32),
            pltpu.VMEM((_NBF, _CH, _DH), jnp.float32),
            pltpu.VMEM((_RCH, _DH), jnp.float32),
            pltpu.VMEM_SHARED((_N + _NTR, _DH), jnp.float32),
            pltpu.SemaphoreType.DMA((_NPAR,)),
            pltpu.SemaphoreType.DMA((_NBF,)),
            pltpu.SemaphoreType.DMA((_NBF,)),
        ],
    )
    def k(zt_hbm, src_hbm, dst_hbm, out_hbm, sidx, didx, rows_v,
          buf_v, agg_sh, isem, gsem, ssem):
        c = lax.axis_index("c")
        s = lax.axis_index("s")
        zc = zt_hbm.at[c]

        def issue_idx(g, par):
            off = g * _NBF * _CH
            pltpu.async_copy(src_hbm.at[pl.ds(s * _EPP + off, _NBF * _CH)],
                             sidx.at[par], isem.at[par])
            pltpu.async_copy(dst_hbm.at[pl.ds(s * _EIT + g * _NBF, _NBF)],
                             didx.at[par], isem.at[par])

        def wait_idx(par):
            pltpu.make_async_copy(src_hbm.at[pl.ds(0, _NBF * _CH)],
                                  sidx.at[par], isem.at[par]).wait()
            pltpu.make_async_copy(dst_hbm.at[pl.ds(0, _NBF)],
                                  didx.at[par], isem.at[par]).wait()

        def ibody(i, carry):
            idx = s + _NS * i

            @pl.when(idx < _NRC)
            def _():
                r0 = idx * _RCH
                pltpu.sync_copy(zc.at[pl.ds(r0, _RCH)], buf_v)
                pltpu.sync_copy(buf_v, agg_sh.at[pl.ds(r0, _RCH)])
            return carry
        lax.fori_loop(0, _RIT, ibody, 0)
        plsc.subcore_barrier()

        def issue_gather(par, slot, b):
            pltpu.async_copy(
                zc.at[sidx.at[par].at[pl.ds(slot * _CH, _CH)]],
                rows_v.at[b], gsem.at[b])

        def wait_gather(b):
            pltpu.make_async_copy(zc.at[sidx.at[0].at[pl.ds(0, _CH)]],
                                  rows_v.at[b], gsem.at[b]).wait()

        def issue_scatter(par, slot, b):
            pltpu.async_copy(rows_v.at[b], agg_sh.at[didx.at[par].at[slot]],
                             ssem.at[b], add=True)

        def wait_scatter(b):
            pltpu.make_async_copy(rows_v.at[b], agg_sh.at[didx.at[0].at[0]],
                                  ssem.at[b]).wait()

        issue_idx(0, 0)
        issue_idx(1, 1)
        issue_idx(2, 2)
        wait_idx(0)
        for b in range(_NBF):
            issue_gather(0, b, b)
        wait_gather(0)
        issue_scatter(0, 0, 0)
        wait_idx(1)
        for b in range(1, _NBF):
            wait_gather(b)
            issue_scatter(0, b, b)
            wait_scatter(b - 1)
            issue_gather(1, b - 1, b - 1)

        def ebody(g, carry):
            par = g % _NPAR
            parn = (g + 1) % _NPAR

            wait_gather(0)
            issue_scatter(par, 0, 0)
            wait_scatter(_NBF - 1)
            issue_gather(par, _NBF - 1, _NBF - 1)

            @pl.when(g < _NGR - 2)
            def _():
                issue_idx(g + 2, (g + 2) % _NPAR)

            @pl.when(g < _NGR - 1)
            def _():
                wait_idx(parn)

            for b in range(1, _NBF):
                wait_gather(b)
                issue_scatter(par, b, b)
                wait_scatter(b - 1)
                issue_gather(parn, b - 1, b - 1)
            return carry
        lax.fori_loop(1, _NGR - 1, ebody, 0)

        parl = (_NGR - 1) % _NPAR
        wait_gather(0)
        issue_scatter(parl, 0, 0)
        wait_scatter(_NBF - 1)
        issue_gather(parl, _NBF - 1, _NBF - 1)
        for b in range(1, _NBF):
            wait_gather(b)
            issue_scatter(parl, b, b)
        for b in range(_NBF):
            wait_scatter(b)
        plsc.subcore_barrier()

        def obody(i, carry):
            idx = s + _NS * i

            @pl.when(idx < _NRC)
            def _():
                r0 = idx * _RCH
                pltpu.sync_copy(agg_sh.at[pl.ds(r0, _RCH)], buf_v)
                pltpu.sync_copy(buf_v, out_hbm.at[pl.ds(c * _N + r0, _RCH)])
            return carry
        lax.fori_loop(0, _RIT, obody, 0)

    return k(zt, src, dst).reshape(_NC, _N, _DH)


def _dot(a, b):
    return lax.dot_general(a, b, (((1,), (1,)), ((), ())),
                           preferred_element_type=jnp.float32)


def _mlp1_body(p_ref, w_ref, b_ref, h_ref, st_ref):
    i = pl.program_id(0)
    a = jnp.concatenate([p_ref[0], p_ref[1]], axis=1)
    h = _dot(a, w_ref[...]) + b_ref[...]
    h_ref[...] = h

    @pl.when(i == 0)
    def _():
        st_ref[...] = jnp.zeros_like(st_ref)
    st_ref[...] += jnp.concatenate(
        [jnp.sum(h, axis=0, keepdims=True),
         jnp.sum(h * h, axis=0, keepdims=True)], axis=0)


def _mlp1(p, w1, b1):
    return pl.pallas_call(
        _mlp1_body,
        grid=(_NB,),
        in_specs=[
            pl.BlockSpec((_NC, _BR, _DH), lambda i: (0, i, 0)),
            pl.BlockSpec((_D, _D), lambda i: (0, 0)),
            pl.BlockSpec((1, _D), lambda i: (0, 0)),
        ],
        out_specs=[
            pl.BlockSpec((_BR, _D), lambda i: (i, 0)),
            pl.BlockSpec((2, _D), lambda i: (0, 0)),
        ],
        out_shape=[
            jax.ShapeDtypeStruct((_N, _D), jnp.float32),
            jax.ShapeDtypeStruct((2, _D), jnp.float32),
        ],
    )(p, w1, b1)


def _bn_coeffs(st_ref, g_ref, be_ref):
    mu = st_ref[0:1, :] * (1.0 / _N)
    var = st_ref[1:2, :] * (1.0 / _N) - mu * mu
    inv = lax.rsqrt(var + _EPS)
    scale = g_ref[...] * inv
    shift = be_ref[...] - mu * scale
    return scale, shift


def _mlp2_body(h_ref, st_ref, g_ref, be_ref, w_ref, b_ref, t_ref, st2_ref):
    i = pl.program_id(0)
    scale, shift = _bn_coeffs(st_ref, g_ref, be_ref)
    hn = jnp.maximum(h_ref[...] * scale + shift, 0.0)
    t = jnp.maximum(_dot(hn, w_ref[...]) + b_ref[...], 0.0)
    t_ref[...] = t

    @pl.when(i == 0)
    def _():
        st2_ref[...] = jnp.zeros_like(st2_ref)
    st2_ref[...] += jnp.concatenate(
        [jnp.sum(t, axis=0, keepdims=True),
         jnp.sum(t * t, axis=0, keepdims=True)], axis=0)


def _mlp2(h, st, g1, be1, w2, b2):
    return pl.pallas_call(
        _mlp2_body,
        grid=(_NB,),
        in_specs=[
            pl.BlockSpec((_BR, _D), lambda i: (i, 0)),
            pl.BlockSpec((2, _D), lambda i: (0, 0)),
            pl.BlockSpec((1, _D), lambda i: (0, 0)),
            pl.BlockSpec((1, _D), lambda i: (0, 0)),
            pl.BlockSpec((_D, _D), lambda i: (0, 0)),
            pl.BlockSpec((1, _D), lambda i: (0, 0)),
        ],
        out_specs=[
            pl.BlockSpec((_BR, _D), lambda i: (i, 0)),
            pl.BlockSpec((2, _D), lambda i: (0, 0)),
        ],
        out_shape=[
            jax.ShapeDtypeStruct((_N, _D), jnp.float32),
            jax.ShapeDtypeStruct((2, _D), jnp.float32),
        ],
    )(h, st, g1, be1, w2, b2)


def _bnpool_body(t_ref, st_ref, g_ref, bo_ref, batch_ref, z_ref, gp_ref):
    i = pl.program_id(0)
    scale, shift = _bn_coeffs(st_ref, g_ref, bo_ref)
    z = t_ref[...] * scale + shift
    z_ref[...] = z

    b = batch_ref[0]
    onehot = (lax.broadcasted_iota(jnp.int32, (_G, _BR), 0) == b
              ).astype(jnp.float32)
    gp = lax.dot_general(onehot, z, (((1,), (0,)), ((), ())),
                         preferred_element_type=jnp.float32)

    @pl.when(i == 0)
    def _():
        gp_ref[...] = jnp.zeros_like(gp_ref)
    gp_ref[...] += gp


def _bnpool(t, st, go, bo, batch3):
    return pl.pallas_call(
        _bnpool_body,
        grid=(_NB,),
        in_specs=[
            pl.BlockSpec((_BR, _D), lambda i: (i, 0)),
            pl.BlockSpec((2, _D), lambda i: (0, 0)),
            pl.BlockSpec((1, _D), lambda i: (0, 0)),
            pl.BlockSpec((1, _D), lambda i: (0, 0)),
            pl.BlockSpec((1, 1, _BR), lambda i: (i, 0, 0)),
        ],
        out_specs=[
            pl.BlockSpec((_BR, _D), lambda i: (i, 0)),
            pl.BlockSpec((_G, _D), lambda i: (0, 0)),
        ],
        out_shape=[
            jax.ShapeDtypeStruct((_N, _D), jnp.float32),
            jax.ShapeDtypeStruct((_G, _D), jnp.float32),
        ],
    )(t, st, go, bo, batch3)


def kernel(x, edge_index, batch,
           W1_0, b1_0, g1_0, be1_0, W2_0, b2_0, go_0, bo_0,
           W1_1, b1_1, g1_1, be1_1, W2_1, b2_1, go_1, bo_1):
    pad = _EPP - _EPT
    src = jnp.pad(edge_index[0].reshape(_NS, _EPT),
                  ((0, 0), (0, pad))).reshape(_NS * _EPP)
    dst = jnp.pad(edge_index[1].reshape(_NS, _EPT), ((0, 0), (0, pad)),
                  constant_values=_N).reshape(_NS * _EIT, _CH)
    batch3 = batch.reshape(_NB, 1, _BR)
    params = [(W1_0, b1_0, g1_0, be1_0, W2_0, b2_0, go_0, bo_0),
              (W1_1, b1_1, g1_1, be1_1, W2_1, b2_1, go_1, bo_1)]

    z = x
    zs, gs = [], []
    for (W1, b1, g1, be1, W2, b2, go, bo) in params:
        zt = jnp.moveaxis(z.reshape(_N, _NC, _DH), 1, 0)
        p = _sc_agg(zt, src, dst)
        h, st1 = _mlp1(p, W1, b1.reshape(1, _D))
        t, st2 = _mlp2(h, st1, g1.reshape(1, _D), be1.reshape(1, _D),
                       W2, b2.reshape(1, _D))
        z, g = _bnpool(t, st2, go.reshape(1, _D), bo.reshape(1, _D), batch3)
        zs.append(z)
        gs.append(g)

    return jnp.concatenate(zs, axis=1), jnp.concatenate(gs, axis=1)


# speedup vs baseline: 8.0525x; 1.5423x over previous
import functools

import jax
import jax.numpy as jnp
from jax import lax
from jax.experimental import pallas as pl
from jax.experimental.pallas import tpu as pltpu
from jax.experimental.pallas import tpu_sc as plsc

_N = 10000
_D = 128
_E = 320000
_G = 64
_NC = 2
_NS = 16
_NW = _NC * _NS
_DH = _D // _NC
_EPT = _E // _NS
_CH = 128
_EIT = 160
_EPP = _EIT * _CH
_NBF = 4
_NGR = _EIT // _NBF
_NPAR = 3
_NTR = 16
_RCH = 100
_NRC = _N // _RCH
_RIT = -(-_NRC // _NS)

_BR = 1000
_NB = _N // _BR
_EPS = 1e-5


def _sc_agg(zt, src, dst):
    mesh = plsc.VectorSubcoreMesh(core_axis_name="c", subcore_axis_name="s",
                                  num_cores=_NC, num_subcores=_NS)

    @functools.partial(
        pl.kernel, mesh=mesh,
        compiler_params=pltpu.CompilerParams(use_tc_tiling_on_sc=False),
        out_type=jax.ShapeDtypeStruct((_NC * _N, _DH), jnp.float32),
        scratch_types=[
            pltpu.VMEM((_NPAR, _NBF * _CH), jnp.int32),
            pltpu.VMEM((_NPAR, _NBF, _CH), jnp.int32),
            pltpu.VMEM((_NBF, _CH, _DH), jnp.float32),
            pltpu.VMEM((_RCH, _DH), jnp.float32),
            pltpu.VMEM_SHARED((_N + _NTR, _DH), jnp.float32),
            pltpu.VMEM_SHARED((_N, _DH), jnp.float32),
            pltpu.SemaphoreType.DMA((_NPAR,)),
            pltpu.SemaphoreType.DMA((_NBF,)),
            pltpu.SemaphoreType.DMA((_NBF,)),
        ],
    )
    def k(zt_hbm, src_hbm, dst_hbm, out_hbm, sidx, didx, rows_v,
          buf_v, agg_sh, z_sh, isem, gsem, ssem):
        c = lax.axis_index("c")
        s = lax.axis_index("s")
        zc = zt_hbm.at[c]

        def issue_idx(g, par):
            off = g * _NBF * _CH
            pltpu.async_copy(src_hbm.at[pl.ds(s * _EPP + off, _NBF * _CH)],
                             sidx.at[par], isem.at[par])
            pltpu.async_copy(dst_hbm.at[pl.ds(s * _EIT + g * _NBF, _NBF)],
                             didx.at[par], isem.at[par])

        def wait_idx(par):
            pltpu.make_async_copy(src_hbm.at[pl.ds(0, _NBF * _CH)],
                                  sidx.at[par], isem.at[par]).wait()
            pltpu.make_async_copy(dst_hbm.at[pl.ds(0, _NBF)],
                                  didx.at[par], isem.at[par]).wait()

        def ibody(i, carry):
            idx = s + _NS * i

            @pl.when(idx < _NRC)
            def _():
                r0 = idx * _RCH
                pltpu.sync_copy(zc.at[pl.ds(r0, _RCH)], buf_v)
                pltpu.sync_copy(buf_v, agg_sh.at[pl.ds(r0, _RCH)])
                pltpu.sync_copy(buf_v, z_sh.at[pl.ds(r0, _RCH)])
            return carry
        lax.fori_loop(0, _RIT, ibody, 0)
        plsc.subcore_barrier()

        def issue_gather(par, slot, b):
            pltpu.async_copy(
                z_sh.at[sidx.at[par].at[pl.ds(slot * _CH, _CH)]],
                rows_v.at[b], gsem.at[b])

        def wait_gather(b):
            pltpu.make_async_copy(z_sh.at[sidx.at[0].at[pl.ds(0, _CH)]],
                                  rows_v.at[b], gsem.at[b]).wait()

        def issue_scatter(par, slot, b):
            pltpu.async_copy(rows_v.at[b], agg_sh.at[didx.at[par].at[slot]],
                             ssem.at[b], add=True)

        def wait_scatter(b):
            pltpu.make_async_copy(rows_v.at[b], agg_sh.at[didx.at[0].at[0]],
                                  ssem.at[b]).wait()

        issue_idx(0, 0)
        issue_idx(1, 1)
        issue_idx(2, 2)
        wait_idx(0)
        for b in range(_NBF):
            issue_gather(0, b, b)
        wait_gather(0)
        issue_scatter(0, 0, 0)
        wait_idx(1)
        for b in range(1, _NBF):
            wait_gather(b)
            issue_scatter(0, b, b)
            wait_scatter(b - 1)
            issue_gather(1, b - 1, b - 1)

        def ebody(g, carry):
            par = g % _NPAR
            parn = (g + 1) % _NPAR

            wait_gather(0)
            issue_scatter(par, 0, 0)
            wait_scatter(_NBF - 1)
            issue_gather(par, _NBF - 1, _NBF - 1)

            @pl.when(g < _NGR - 2)
            def _():
                issue_idx(g + 2, (g + 2) % _NPAR)

            @pl.when(g < _NGR - 1)
            def _():
                wait_idx(parn)

            for b in range(1, _NBF):
                wait_gather(b)
                issue_scatter(par, b, b)
                wait_scatter(b - 1)
                issue_gather(parn, b - 1, b - 1)
            return carry
        lax.fori_loop(1, _NGR - 1, ebody, 0)

        parl = (_NGR - 1) % _NPAR
        wait_gather(0)
        issue_scatter(parl, 0, 0)
        wait_scatter(_NBF - 1)
        issue_gather(parl, _NBF - 1, _NBF - 1)
        for b in range(1, _NBF):
            wait_gather(b)
            issue_scatter(parl, b, b)
        for b in range(_NBF):
            wait_scatter(b)
        plsc.subcore_barrier()

        def obody(i, carry):
            idx = s + _NS * i

            @pl.when(idx < _NRC)
            def _():
                r0 = idx * _RCH
                pltpu.sync_copy(agg_sh.at[pl.ds(r0, _RCH)], buf_v)
                pltpu.sync_copy(buf_v, out_hbm.at[pl.ds(c * _N + r0, _RCH)])
            return carry
        lax.fori_loop(0, _RIT, obody, 0)

    return k(zt, src, dst).reshape(_NC, _N, _DH)


def _dot(a, b):
    return lax.dot_general(a, b, (((1,), (1,)), ((), ())),
                           preferred_element_type=jnp.float32)


def _mlp1_body(p_ref, w_ref, b_ref, h_ref, st_ref):
    i = pl.program_id(0)
    a = jnp.concatenate([p_ref[0], p_ref[1]], axis=1)
    h = _dot(a, w_ref[...]) + b_ref[...]
    h_ref[...] = h

    @pl.when(i == 0)
    def _():
        st_ref[...] = jnp.zeros_like(st_ref)
    st_ref[...] += jnp.concatenate(
        [jnp.sum(h, axis=0, keepdims=True),
         jnp.sum(h * h, axis=0, keepdims=True)], axis=0)


def _mlp1(p, w1, b1):
    return pl.pallas_call(
        _mlp1_body,
        grid=(_NB,),
        in_specs=[
            pl.BlockSpec((_NC, _BR, _DH), lambda i: (0, i, 0)),
            pl.BlockSpec((_D, _D), lambda i: (0, 0)),
            pl.BlockSpec((1, _D), lambda i: (0, 0)),
        ],
        out_specs=[
            pl.BlockSpec((_BR, _D), lambda i: (i, 0)),
            pl.BlockSpec((2, _D), lambda i: (0, 0)),
        ],
        out_shape=[
            jax.ShapeDtypeStruct((_N, _D), jnp.float32),
            jax.ShapeDtypeStruct((2, _D), jnp.float32),
        ],
    )(p, w1, b1)


def _bn_coeffs(st_ref, g_ref, be_ref):
    mu = st_ref[0:1, :] * (1.0 / _N)
    var = st_ref[1:2, :] * (1.0 / _N) - mu * mu
    inv = lax.rsqrt(var + _EPS)
    scale = g_ref[...] * inv
    shift = be_ref[...] - mu * scale
    return scale, shift


def _mlp2_body(h_ref, st_ref, g_ref, be_ref, w_ref, b_ref, t_ref, st2_ref):
    i = pl.program_id(0)
    scale, shift = _bn_coeffs(st_ref, g_ref, be_ref)
    hn = jnp.maximum(h_ref[...] * scale + shift, 0.0)
    t = jnp.maximum(_dot(hn, w_ref[...]) + b_ref[...], 0.0)
    t_ref[...] = t

    @pl.when(i == 0)
    def _():
        st2_ref[...] = jnp.zeros_like(st2_ref)
    st2_ref[...] += jnp.concatenate(
        [jnp.sum(t, axis=0, keepdims=True),
         jnp.sum(t * t, axis=0, keepdims=True)], axis=0)


def _mlp2(h, st, g1, be1, w2, b2):
    return pl.pallas_call(
        _mlp2_body,
        grid=(_NB,),
        in_specs=[
            pl.BlockSpec((_BR, _D), lambda i: (i, 0)),
            pl.BlockSpec((2, _D), lambda i: (0, 0)),
            pl.BlockSpec((1, _D), lambda i: (0, 0)),
            pl.BlockSpec((1, _D), lambda i: (0, 0)),
            pl.BlockSpec((_D, _D), lambda i: (0, 0)),
            pl.BlockSpec((1, _D), lambda i: (0, 0)),
        ],
        out_specs=[
            pl.BlockSpec((_BR, _D), lambda i: (i, 0)),
            pl.BlockSpec((2, _D), lambda i: (0, 0)),
        ],
        out_shape=[
            jax.ShapeDtypeStruct((_N, _D), jnp.float32),
            jax.ShapeDtypeStruct((2, _D), jnp.float32),
        ],
    )(h, st, g1, be1, w2, b2)


def _bnpool_body(t_ref, st_ref, g_ref, bo_ref, batch_ref, z_ref, gp_ref):
    i = pl.program_id(0)
    scale, shift = _bn_coeffs(st_ref, g_ref, bo_ref)
    z = t_ref[...] * scale + shift
    z_ref[...] = z

    b = batch_ref[0]
    onehot = (lax.broadcasted_iota(jnp.int32, (_G, _BR), 0) == b
              ).astype(jnp.float32)
    gp = lax.dot_general(onehot, z, (((1,), (0,)), ((), ())),
                         preferred_element_type=jnp.float32)

    @pl.when(i == 0)
    def _():
        gp_ref[...] = jnp.zeros_like(gp_ref)
    gp_ref[...] += gp


def _bnpool(t, st, go, bo, batch3):
    return pl.pallas_call(
        _bnpool_body,
        grid=(_NB,),
        in_specs=[
            pl.BlockSpec((_BR, _D), lambda i: (i, 0)),
            pl.BlockSpec((2, _D), lambda i: (0, 0)),
            pl.BlockSpec((1, _D), lambda i: (0, 0)),
            pl.BlockSpec((1, _D), lambda i: (0, 0)),
            pl.BlockSpec((1, 1, _BR), lambda i: (i, 0, 0)),
        ],
        out_specs=[
            pl.BlockSpec((_BR, _D), lambda i: (i, 0)),
            pl.BlockSpec((_G, _D), lambda i: (0, 0)),
        ],
        out_shape=[
            jax.ShapeDtypeStruct((_N, _D), jnp.float32),
            jax.ShapeDtypeStruct((_G, _D), jnp.float32),
        ],
    )(t, st, go, bo, batch3)


def kernel(x, edge_index, batch,
           W1_0, b1_0, g1_0, be1_0, W2_0, b2_0, go_0, bo_0,
           W1_1, b1_1, g1_1, be1_1, W2_1, b2_1, go_1, bo_1):
    pad = _EPP - _EPT
    src = jnp.pad(edge_index[0].reshape(_NS, _EPT),
                  ((0, 0), (0, pad))).reshape(_NS * _EPP)
    dst = jnp.pad(edge_index[1].reshape(_NS, _EPT), ((0, 0), (0, pad)),
                  constant_values=_N).reshape(_NS * _EIT, _CH)
    batch3 = batch.reshape(_NB, 1, _BR)
    params = [(W1_0, b1_0, g1_0, be1_0, W2_0, b2_0, go_0, bo_0),
              (W1_1, b1_1, g1_1, be1_1, W2_1, b2_1, go_1, bo_1)]

    z = x
    zs, gs = [], []
    for (W1, b1, g1, be1, W2, b2, go, bo) in params:
        zt = jnp.moveaxis(z.reshape(_N, _NC, _DH), 1, 0)
        p = _sc_agg(zt, src, dst)
        h, st1 = _mlp1(p, W1, b1.reshape(1, _D))
        t, st2 = _mlp2(h, st1, g1.reshape(1, _D), be1.reshape(1, _D),
                       W2, b2.reshape(1, _D))
        z, g = _bnpool(t, st2, go.reshape(1, _D), bo.reshape(1, _D), batch3)
        zs.append(z)
        gs.append(g)

    return jnp.concatenate(zs, axis=1), jnp.concatenate(gs, axis=1)


# speedup vs baseline: 8.5403x; 1.0606x over previous
import functools

import jax
import jax.numpy as jnp
from jax import lax
from jax.experimental import pallas as pl
from jax.experimental.pallas import tpu as pltpu
from jax.experimental.pallas import tpu_sc as plsc

_N = 10000
_D = 128
_E = 320000
_G = 64
_NC = 2
_NS = 16
_NW = _NC * _NS
_DH = _D // _NC
_EPT = _E // _NS
_CH = 128
_EIT = 160
_EPP = _EIT * _CH
_NBF = 4
_NGR = _EIT // _NBF
_NPAR = 3
_NTR = 16
_RCH = 100
_NRC = _N // _RCH
_RIT = -(-_NRC // _NS)

_BR = 1000
_NB = _N // _BR
_EPS = 1e-5


def _sc_agg(zt, src, dst):
    mesh = plsc.VectorSubcoreMesh(core_axis_name="c", subcore_axis_name="s",
                                  num_cores=_NC, num_subcores=_NS)

    @functools.partial(
        pl.kernel, mesh=mesh,
        compiler_params=pltpu.CompilerParams(use_tc_tiling_on_sc=False),
        out_type=jax.ShapeDtypeStruct((_NC * _N, _DH), jnp.float32),
        scratch_types=[
            pltpu.VMEM((_NPAR, _NBF * _CH), jnp.int32),
            pltpu.VMEM((_NPAR, _NBF, _CH), jnp.int32),
            pltpu.VMEM((_NBF, _CH, _DH), jnp.float32),
            pltpu.VMEM((_RCH, _DH), jnp.float32),
            pltpu.VMEM_SHARED((_N + _NTR, _DH), jnp.float32),
            pltpu.VMEM_SHARED((_N, _DH), jnp.float32),
            pltpu.SemaphoreType.DMA((_NPAR,)),
            pltpu.SemaphoreType.DMA((_NBF,)),
            pltpu.SemaphoreType.DMA((_NBF,)),
        ],
    )
    def k(zt_hbm, src_hbm, dst_hbm, out_hbm, sidx, didx, rows_v,
          buf_v, agg_sh, z_sh, isem, gsem, ssem):
        c = lax.axis_index("c")
        s = lax.axis_index("s")
        zc = zt_hbm.at[c]

        def issue_idx(g, par):
            off = g * _NBF * _CH
            pltpu.async_copy(src_hbm.at[pl.ds(s * _EPP + off, _NBF * _CH)],
                             sidx.at[par], isem.at[par])
            pltpu.async_copy(dst_hbm.at[pl.ds(s * _EIT + g * _NBF, _NBF)],
                             didx.at[par], isem.at[par])

        def wait_idx(par):
            pltpu.make_async_copy(src_hbm.at[pl.ds(0, _NBF * _CH)],
                                  sidx.at[par], isem.at[par]).wait()
            pltpu.make_async_copy(dst_hbm.at[pl.ds(0, _NBF)],
                                  didx.at[par], isem.at[par]).wait()

        def ibody(i, carry):
            idx = s + _NS * i

            @pl.when(idx < _NRC)
            def _():
                r0 = idx * _RCH
                pltpu.sync_copy(zc.at[pl.ds(r0, _RCH)], buf_v)
                pltpu.sync_copy(buf_v, agg_sh.at[pl.ds(r0, _RCH)])
                pltpu.sync_copy(buf_v, z_sh.at[pl.ds(r0, _RCH)])
            return carry
        lax.fori_loop(0, _RIT, ibody, 0)
        plsc.subcore_barrier()

        def issue_gather(par, slot, b):
            pltpu.async_copy(
                z_sh.at[sidx.at[par].at[pl.ds(slot * _CH, _CH)]],
                rows_v.at[b], gsem.at[b])

        def wait_gather(b):
            pltpu.make_async_copy(z_sh.at[sidx.at[0].at[pl.ds(0, _CH)]],
                                  rows_v.at[b], gsem.at[b]).wait()

        def issue_scatter(par, slot, b):
            pltpu.async_copy(rows_v.at[b], agg_sh.at[didx.at[par].at[slot]],
                             ssem.at[b], add=True)

        def wait_scatter(b):
            pltpu.make_async_copy(rows_v.at[b], agg_sh.at[didx.at[0].at[0]],
                                  ssem.at[b]).wait()

        issue_idx(0, 0)
        issue_idx(1, 1)
        issue_idx(2, 2)
        wait_idx(0)
        for b in range(_NBF):
            issue_gather(0, b, b)
        wait_gather(0)
        issue_scatter(0, 0, 0)
        wait_idx(1)
        for b in range(1, _NBF):
            wait_gather(b)
            issue_scatter(0, b, b)
            wait_scatter(b - 1)
            issue_gather(1, b - 1, b - 1)

        def ebody(g, carry):
            par = g % _NPAR
            parn = (g + 1) % _NPAR

            wait_gather(0)
            issue_scatter(par, 0, 0)
            wait_scatter(_NBF - 1)
            issue_gather(par, _NBF - 1, _NBF - 1)

            @pl.when(g < _NGR - 2)
            def _():
                issue_idx(g + 2, (g + 2) % _NPAR)

            @pl.when(g < _NGR - 1)
            def _():
                wait_idx(parn)

            for b in range(1, _NBF):
                wait_gather(b)
                issue_scatter(par, b, b)
                wait_scatter(b - 1)
                issue_gather(parn, b - 1, b - 1)
            return carry
        lax.fori_loop(1, _NGR - 1, ebody, 0)

        parl = (_NGR - 1) % _NPAR
        wait_gather(0)
        issue_scatter(parl, 0, 0)
        wait_scatter(_NBF - 1)
        issue_gather(parl, _NBF - 1, _NBF - 1)
        for b in range(1, _NBF):
            wait_gather(b)
            issue_scatter(parl, b, b)
        for b in range(_NBF):
            wait_scatter(b)
        plsc.subcore_barrier()

        def obody(i, carry):
            idx = s + _NS * i

            @pl.when(idx < _NRC)
            def _():
                r0 = idx * _RCH
                pltpu.sync_copy(agg_sh.at[pl.ds(r0, _RCH)], buf_v)
                pltpu.sync_copy(buf_v, out_hbm.at[pl.ds(c * _N + r0, _RCH)])
            return carry
        lax.fori_loop(0, _RIT, obody, 0)

    return k(zt, src, dst).reshape(_NC, _N, _DH)


def _dot(a, b):
    return lax.dot_general(a, b, (((1,), (1,)), ((), ())),
                           preferred_element_type=jnp.float32)


def _bn_coeffs(st, g_ref, be_ref):
    mu = st[0:1, :] * (1.0 / _N)
    var = st[1:2, :] * (1.0 / _N) - mu * mu
    inv = lax.rsqrt(var + _EPS)
    scale = g_ref[...] * inv
    shift = be_ref[...] - mu * scale
    return scale, shift


def _stats(x):
    return jnp.concatenate([jnp.sum(x, axis=0, keepdims=True),
                            jnp.sum(x * x, axis=0, keepdims=True)], axis=0)


def _layer_body(p_ref, w1_ref, b1_ref, g1_ref, be1_ref, w2_ref, b2_ref,
                go_ref, bo_ref, batch_ref, z_ref, zt_ref, gp_ref,
                h_scr, t_scr, st1, st2):
    ph = pl.program_id(0)
    i = pl.program_id(1)
    rows = pl.ds(i * _BR, _BR)

    @pl.when(jnp.logical_and(ph == 0, i == 0))
    def _():
        st1[...] = jnp.zeros_like(st1)
        st2[...] = jnp.zeros_like(st2)
        gp_ref[...] = jnp.zeros_like(gp_ref)

    @pl.when(ph == 0)
    def _():
        a = jnp.concatenate([p_ref[0], p_ref[1]], axis=1)
        h = _dot(a, w1_ref[...]) + b1_ref[...]
        h_scr[rows, :] = h
        st1[...] += _stats(h)

    @pl.when(ph == 1)
    def _():
        scale, shift = _bn_coeffs(st1[...], g1_ref, be1_ref)
        hn = jnp.maximum(h_scr[rows, :] * scale + shift, 0.0)
        t = jnp.maximum(_dot(hn, w2_ref[...]) + b2_ref[...], 0.0)
        t_scr[rows, :] = t
        st2[...] += _stats(t)

    @pl.when(ph == 2)
    def _():
        scale, shift = _bn_coeffs(st2[...], go_ref, bo_ref)
        z = t_scr[rows, :] * scale + shift
        z_ref[...] = z
        zt_ref[0] = z[:, :_DH]
        zt_ref[1] = z[:, _DH:]
        b = batch_ref[0]
        onehot = (lax.broadcasted_iota(jnp.int32, (_G, _BR), 0) == b
                  ).astype(jnp.float32)
        gp_ref[...] += lax.dot_general(onehot, z, (((1,), (0,)), ((), ())),
                                       preferred_element_type=jnp.float32)


def _layer(p, w1, b1, g1, be1, w2, b2, go, bo, batch3):
    cparam = pl.BlockSpec((1, _D), lambda ph, i: (0, 0))
    return pl.pallas_call(
        _layer_body,
        grid=(3, _NB),
        in_specs=[
            pl.BlockSpec((_NC, _BR, _DH),
                         lambda ph, i: (0, i * (ph == 0), 0)),
            pl.BlockSpec((_D, _D), lambda ph, i: (0, 0)),
            cparam, cparam, cparam,
            pl.BlockSpec((_D, _D), lambda ph, i: (0, 0)),
            cparam, cparam, cparam,
            pl.BlockSpec((1, 1, _BR), lambda ph, i: (i * (ph == 2), 0, 0)),
        ],
        out_specs=[
            pl.BlockSpec((_BR, _D), lambda ph, i: (i * (ph == 2), 0)),
            pl.BlockSpec((_NC, _BR, _DH),
                         lambda ph, i: (0, i * (ph == 2), 0)),
            pl.BlockSpec((_G, _D), lambda ph, i: (0, 0)),
        ],
        out_shape=[
            jax.ShapeDtypeStruct((_N, _D), jnp.float32),
            jax.ShapeDtypeStruct((_NC, _N, _DH), jnp.float32),
            jax.ShapeDtypeStruct((_G, _D), jnp.float32),
        ],
        scratch_shapes=[
            pltpu.VMEM((_N, _D), jnp.float32),
            pltpu.VMEM((_N, _D), jnp.float32),
            pltpu.VMEM((2, _D), jnp.float32),
            pltpu.VMEM((2, _D), jnp.float32),
        ],
    )(p, w1, b1, g1, be1, w2, b2, go, bo, batch3)


def kernel(x, edge_index, batch,
           W1_0, b1_0, g1_0, be1_0, W2_0, b2_0, go_0, bo_0,
           W1_1, b1_1, g1_1, be1_1, W2_1, b2_1, go_1, bo_1):
    pad = _EPP - _EPT
    src = jnp.pad(edge_index[0].reshape(_NS, _EPT),
                  ((0, 0), (0, pad))).reshape(_NS * _EPP)
    dst = jnp.pad(edge_index[1].reshape(_NS, _EPT), ((0, 0), (0, pad)),
                  constant_values=_N).reshape(_NS * _EIT, _CH)
    batch3 = batch.reshape(_NB, 1, _BR)
    params = [(W1_0, b1_0, g1_0, be1_0, W2_0, b2_0, go_0, bo_0),
              (W1_1, b1_1, g1_1, be1_1, W2_1, b2_1, go_1, bo_1)]

    zt = jnp.moveaxis(x.reshape(_N, _NC, _DH), 1, 0)
    zs, gs = [], []
    for (W1, b1, g1, be1, W2, b2, go, bo) in params:
        p = _sc_agg(zt, src, dst)
        z, zt, g = _layer(p, W1, b1.reshape(1, _D), g1.reshape(1, _D),
                          be1.reshape(1, _D), W2, b2.reshape(1, _D),
                          go.reshape(1, _D), bo.reshape(1, _D), batch3)
        zs.append(z)
        gs.append(g)

    return jnp.concatenate(zs, axis=1), jnp.concatenate(gs, axis=1)
